# baseline (speedup 1.0000x reference)
import functools

import jax
import jax.numpy as jnp
import numpy as np
from jax import lax
from jax.experimental import pallas as pl
from jax.experimental.pallas import tpu as pltpu
from jax.experimental.pallas import tpu_sc as plsc

B = 4096
F = 26
V = 1000000
E = 16
D = 4
H1 = 512
H2 = 128
EMB = F * E

NC = 2
NS = 16
NW = NC * NS
BF = B * F
CHUNK = 128
NCHUNK = BF // (NW * CHUNK)

TB = 512
NT = B // TB


NSLAB = 4
NGRAN = NSLAB * B * 128 // E


def _gather_body(idx_hbm, d0_hbm, table_hbm, out_hbm, idx_v, didx_v, rows_v,
                 sem):
    c = lax.axis_index("c")
    s = lax.axis_index("s")
    wid = s * NC + c
    pltpu.sync_copy(idx_hbm.at[wid], idx_v)
    pltpu.sync_copy(d0_hbm, didx_v)
    woff = lax.shift_left(wid, 10)
    def remap(t, carry):
        j = t // 8
        k = t % 8
        v = idx_v[j, pl.ds(k * 16, 16)]
        p = (lax.shift_left(lax.shift_right_logical(v, 10), 10)
             + lax.shift_left(lax.bitwise_and(v, 127), 3)
             + lax.bitwise_and(lax.shift_right_logical(v, 7), 7))
        idx_v[j, pl.ds(k * 16, 16)] = p
        didx_v[j, pl.ds(k * 16, 16)] = didx_v[j, pl.ds(k * 16, 16)] + woff
        return carry
    lax.fori_loop(0, NCHUNK * 8, remap, 0)
    for j in range(NCHUNK):
        pltpu.async_copy(table_hbm.at[idx_v.at[j]], rows_v.at[j], sem)
    def drain(j, carry):
        pltpu.make_async_copy(table_hbm.at[idx_v.at[0]], rows_v.at[0], sem).wait()
        return carry
    lax.fori_loop(0, NCHUNK, drain, 0)
    for j in range(NCHUNK):
        pltpu.async_copy(rows_v.at[j], out_hbm.at[didx_v.at[j]], sem)
    def drain2(j, carry):
        pltpu.make_async_copy(rows_v.at[0], out_hbm.at[didx_v.at[0]], sem).wait()
        return carry
    lax.fori_loop(0, NCHUNK, drain2, 0)


@functools.partial(jax.jit, static_argnums=())
def _sc_gather(idx, d0, table):
    run = pl.kernel(
        _gather_body,
        out_type=jax.ShapeDtypeStruct((NGRAN, E), jnp.float32),
        mesh=plsc.VectorSubcoreMesh(
            core_axis_name="c", subcore_axis_name="s",
            num_cores=NC, num_subcores=NS),
        scratch_types=[
            pltpu.VMEM((NCHUNK, CHUNK), jnp.int32),
            pltpu.VMEM((NCHUNK, CHUNK), jnp.int32),
            pltpu.VMEM((NCHUNK, CHUNK, E), jnp.float32),
            pltpu.SemaphoreType.DMA,
        ],
        compiler_params=pltpu.CompilerParams(use_tc_tiling_on_sc=False),
    )
    return run(idx, d0, table)


TCOLS = 65536
TGRID = -(-V // TCOLS)


def _tr_body(tt_ref, out_ref):
    v = tt_ref[...]
    for gg in range(TCOLS // 1024):
        m = jnp.concatenate(
            [v[:, (8 * gg + a) * 128:(8 * gg + a + 1) * 128]
             for a in range(8)], axis=0)
        out_ref[gg * 128:(gg + 1) * 128, :] = m.T


def _tc_transpose(tableT):
    return pl.pallas_call(
        _tr_body,
        grid=(TGRID,),
        in_specs=[pl.BlockSpec((E, TCOLS), lambda i: (0, i))],
        out_specs=pl.BlockSpec((TCOLS * E // 128, 128), lambda i: (i, 0)),
        out_shape=jax.ShapeDtypeStruct((TGRID * TCOLS * E // 128, 128),
                                       jnp.float32),
        compiler_params=pltpu.CompilerParams(
            dimension_semantics=("arbitrary",)),
    )(tableT)


def _mlp_body(emb_ref, did_ref, dt_ref, W1e_ref, W1d_ref, b1_ref, W2_ref,
              b2_ref, W3r_ref, b3r_ref, Wd1_ref, bd1_ref, Wd2_ref, bd2_ref,
              Wo_ref, bo_ref, out_ref):
    did = did_ref[0, 0, :]
    onehot = (did[:, None] ==
              lax.broadcasted_iota(jnp.int32, (TB, D), 1)).astype(jnp.float32)
    dt = dt_ref[...]
    wh = jnp.maximum(dt @ Wd1_ref[...] + bd1_ref[...], 0.0)
    wh = jnp.maximum(wh @ Wd2_ref[...] + bd2_ref[...], 0.0)
    logits = wh @ Wo_ref[...] + bo_ref[...]
    m = jnp.max(logits, axis=1, keepdims=True)
    ex = jnp.exp(logits - m)
    wtab = ex / jnp.sum(ex, axis=1, keepdims=True)
    wt = onehot @ wtab

    p0 = emb_ref[0].astype(jnp.bfloat16)
    p1 = emb_ref[1].astype(jnp.bfloat16)
    p2 = emb_ref[2].astype(jnp.bfloat16)
    p3 = emb_ref[3][:, :32].astype(jnp.bfloat16)
    total = jnp.zeros((TB,), jnp.float32)
    for d in range(D):
        dtab = dt @ W1d_ref[d] + b1_ref[d]
        w1 = W1e_ref[d].astype(jnp.bfloat16)
        h1 = jnp.maximum(
            jnp.dot(p0, w1[0:128], preferred_element_type=jnp.float32)
            + jnp.dot(p1, w1[128:256], preferred_element_type=jnp.float32)
            + jnp.dot(p2, w1[256:384], preferred_element_type=jnp.float32)
            + jnp.dot(p3, w1[384:416], preferred_element_type=jnp.float32)
            + onehot @ dtab, 0.0)
        h2 = jnp.maximum(
            jnp.dot(h1.astype(jnp.bfloat16), W2_ref[d].astype(jnp.bfloat16),
                    preferred_element_type=jnp.float32)
            + b2_ref[d], 0.0)
        o = jnp.sum(h2 * W3r_ref[d], axis=1) + b3r_ref[d]
        total = total + o * wt[:, d]
    out_ref[0, 0, :] = 1.0 / (1.0 + jnp.exp(-total))


def _tc_mlp(emb, did3, domain_table, W1e, W1d, b1, W2, b2, W3r, b3r,
            Wd1, bd1, Wd2, bd2, Wo, bo):
    full = lambda *shape: pl.BlockSpec(shape, lambda i: (0,) * len(shape))
    out = pl.pallas_call(
        _mlp_body,
        grid=(NT,),
        in_specs=[
            pl.BlockSpec((NSLAB, TB, 128), lambda i: (0, i, 0)),
            pl.BlockSpec((1, 1, TB), lambda i: (i, 0, 0)),
            full(D, E),
            full(D, EMB, H1),
            full(D, E, H1),
            full(D, H1),
            full(D, H1, H2),
            full(D, H2),
            full(D, H2),
            full(D),
            full(E, 64),
            full(64),
            full(64, D),
            full(D),
            full(D, D),
            full(D),
        ],
        out_specs=pl.BlockSpec((1, 1, TB), lambda i: (i, 0, 0)),
        out_shape=jax.ShapeDtypeStruct((NT, 1, TB), jnp.float32),
        compiler_params=pltpu.CompilerParams(
            dimension_semantics=("arbitrary",)),
    )(emb, did3, domain_table, W1e, W1d, b1, W2, b2, W3r, b3r,
      Wd1, bd1, Wd2, bd2, Wo, bo)
    return out.reshape(B)


def kernel(x, domain_id, table, domain_table, W1, b1, W2, b2, W3, b3,
           Wd1, bd1, Wd2, bd2, Wo, bo):
    idx = x.astype(jnp.int32).reshape(NW, NCHUNK, CHUNK)
    tbl = _tc_transpose(table.T)
    tbl = jnp.reshape(tbl, (TGRID * TCOLS, E))
    flat = np.arange(NCHUNK * CHUNK, dtype=np.int64).reshape(NCHUNK, CHUNK)
    bl, fl = flat // F, flat % F
    d0 = jnp.asarray(((fl >> 3) << 15) + (bl << 3) + (fl & 7), jnp.int32)
    rows = _sc_gather(idx, d0, tbl)
    emb4 = jnp.reshape(rows, (NSLAB, B, 128))
    did3 = domain_id.astype(jnp.int32).reshape(NT, 1, TB)
    W1e = W1[:, :EMB, :]
    W1d = W1[:, EMB:, :]
    W3r = W3[:, :, 0]
    b3r = b3[:, 0]
    return _tc_mlp(emb4, did3, domain_table, W1e, W1d, b1, W2, b2, W3r, b3r,
                   Wd1, bd1, Wd2, bd2, Wo, bo)


# speedup vs baseline: 1.0990x; 1.0990x over previous
import functools

import jax
import jax.numpy as jnp
import numpy as np
from jax import lax
from jax.experimental import pallas as pl
from jax.experimental.pallas import tpu as pltpu
from jax.experimental.pallas import tpu_sc as plsc

B = 4096
F = 26
V = 1000000
E = 16
D = 4
H1 = 512
H2 = 128
EMB = F * E

NC = 2
NS = 16
NW = NC * NS
BF = B * F
CHUNK = 128
NCHUNK = BF // (NW * CHUNK)

TB = 512
NT = B // TB


NSLAB = 4
NGRAN = NSLAB * B * 128 // E


def _gather_body(idx_hbm, d0_hbm, table_hbm, out_hbm, idx_v, didx_v, rows_v,
                 sem):
    c = lax.axis_index("c")
    s = lax.axis_index("s")
    wid = s * NC + c
    pltpu.sync_copy(idx_hbm.at[wid], idx_v)
    pltpu.sync_copy(d0_hbm, didx_v)
    woff = lax.shift_left(wid, 12)
    def remap(t, carry):
        j = t // 8
        k = t % 8
        v = idx_v[j, pl.ds(k * 16, 16)]
        p = (lax.shift_left(lax.shift_right_logical(v, 10), 10)
             + lax.shift_left(lax.bitwise_and(v, 127), 3)
             + lax.bitwise_and(lax.shift_right_logical(v, 7), 7))
        idx_v[j, pl.ds(k * 16, 16)] = p
        didx_v[j, pl.ds(k * 16, 16)] = didx_v[j, pl.ds(k * 16, 16)] + woff
        return carry
    lax.fori_loop(0, NCHUNK * 8, remap, 0)
    for j in range(NCHUNK):
        pltpu.async_copy(table_hbm.at[idx_v.at[j]], rows_v.at[j], sem)
    def drain(j, carry):
        pltpu.make_async_copy(table_hbm.at[idx_v.at[0]], rows_v.at[0], sem).wait()
        return carry
    lax.fori_loop(0, NCHUNK, drain, 0)
    for j in range(NCHUNK):
        pltpu.async_copy(rows_v.at[j], out_hbm.at[didx_v.at[j]], sem)
    def drain2(j, carry):
        pltpu.make_async_copy(rows_v.at[0], out_hbm.at[didx_v.at[0]], sem).wait()
        return carry
    lax.fori_loop(0, NCHUNK, drain2, 0)


@functools.partial(jax.jit, static_argnums=())
def _sc_gather(idx, d0, table):
    run = pl.kernel(
        _gather_body,
        out_type=jax.ShapeDtypeStruct((NGRAN, E), jnp.float32),
        mesh=plsc.VectorSubcoreMesh(
            core_axis_name="c", subcore_axis_name="s",
            num_cores=NC, num_subcores=NS),
        scratch_types=[
            pltpu.VMEM((NCHUNK, CHUNK), jnp.int32),
            pltpu.VMEM((NCHUNK, CHUNK), jnp.int32),
            pltpu.VMEM((NCHUNK, CHUNK, E), jnp.float32),
            pltpu.SemaphoreType.DMA,
        ],
        compiler_params=pltpu.CompilerParams(use_tc_tiling_on_sc=False),
    )
    return run(idx, d0, table)


TCOLS = 65536
TGRID = -(-V // TCOLS)


def _tr_body(tt_ref, out_ref):
    v = tt_ref[...]
    for gg in range(TCOLS // 1024):
        m = jnp.concatenate(
            [v[:, (8 * gg + a) * 128:(8 * gg + a + 1) * 128]
             for a in range(8)], axis=0)
        out_ref[gg * 128:(gg + 1) * 128, :] = m.T


def _tc_transpose(tableT):
    return pl.pallas_call(
        _tr_body,
        grid=(TGRID,),
        in_specs=[pl.BlockSpec((E, TCOLS), lambda i: (0, i))],
        out_specs=pl.BlockSpec((TCOLS * E // 128, 128), lambda i: (i, 0)),
        out_shape=jax.ShapeDtypeStruct((TGRID * TCOLS * E // 128, 128),
                                       jnp.float32),
        compiler_params=pltpu.CompilerParams(
            dimension_semantics=("arbitrary",)),
    )(tableT)


def _mlp_body(emb_ref, did_ref, dt_ref, W1e_ref, W1d_ref, b1_ref, W2_ref,
              b2_ref, W3r_ref, b3r_ref, Wd1_ref, bd1_ref, Wd2_ref, bd2_ref,
              Wo_ref, bo_ref, out_ref):
    did = did_ref[0, 0, :]
    onehot = (did[:, None] ==
              lax.broadcasted_iota(jnp.int32, (TB, D), 1)).astype(jnp.float32)
    dt = dt_ref[...]
    wh = jnp.maximum(dt @ Wd1_ref[...] + bd1_ref[...], 0.0)
    wh = jnp.maximum(wh @ Wd2_ref[...] + bd2_ref[...], 0.0)
    logits = wh @ Wo_ref[...] + bo_ref[...]
    m = jnp.max(logits, axis=1, keepdims=True)
    ex = jnp.exp(logits - m)
    wtab = ex / jnp.sum(ex, axis=1, keepdims=True)
    wt = onehot @ wtab

    blk = emb_ref[...]
    emb = jnp.concatenate(
        [blk[:, t].reshape(TB, 128) for t in range(NSLAB)], axis=1)
    embh = emb[:, :EMB].astype(jnp.bfloat16)
    total = jnp.zeros((TB,), jnp.float32)
    for d in range(D):
        dtab = dt @ W1d_ref[d] + b1_ref[d]
        h1 = jnp.maximum(
            jnp.dot(embh, W1e_ref[d].astype(jnp.bfloat16),
                    preferred_element_type=jnp.float32)
            + onehot @ dtab, 0.0)
        h2 = jnp.maximum(
            jnp.dot(h1.astype(jnp.bfloat16), W2_ref[d].astype(jnp.bfloat16),
                    preferred_element_type=jnp.float32)
            + b2_ref[d], 0.0)
        o = jnp.sum(h2 * W3r_ref[d], axis=1) + b3r_ref[d]
        total = total + o * wt[:, d]
    out_ref[0, 0, :] = 1.0 / (1.0 + jnp.exp(-total))


def _tc_mlp(emb, did3, domain_table, W1e, W1d, b1, W2, b2, W3r, b3r,
            Wd1, bd1, Wd2, bd2, Wo, bo):
    full = lambda *shape: pl.BlockSpec(shape, lambda i: (0,) * len(shape))
    out = pl.pallas_call(
        _mlp_body,
        grid=(NT,),
        in_specs=[
            pl.BlockSpec((TB // 8, NSLAB, 8, 128), lambda i: (i, 0, 0, 0)),
            pl.BlockSpec((1, 1, TB), lambda i: (i, 0, 0)),
            full(D, E),
            full(D, EMB, H1),
            full(D, E, H1),
            full(D, H1),
            full(D, H1, H2),
            full(D, H2),
            full(D, H2),
            full(D),
            full(E, 64),
            full(64),
            full(64, D),
            full(D),
            full(D, D),
            full(D),
        ],
        out_specs=pl.BlockSpec((1, 1, TB), lambda i: (i, 0, 0)),
        out_shape=jax.ShapeDtypeStruct((NT, 1, TB), jnp.float32),
        compiler_params=pltpu.CompilerParams(
            dimension_semantics=("arbitrary",)),
    )(emb, did3, domain_table, W1e, W1d, b1, W2, b2, W3r, b3r,
      Wd1, bd1, Wd2, bd2, Wo, bo)
    return out.reshape(B)


def kernel(x, domain_id, table, domain_table, W1, b1, W2, b2, W3, b3,
           Wd1, bd1, Wd2, bd2, Wo, bo):
    idx = x.astype(jnp.int32).reshape(NW, NCHUNK, CHUNK)
    tbl = _tc_transpose(table.T)
    tbl = jnp.reshape(tbl, (TGRID * TCOLS, E))
    flat = np.arange(NCHUNK * CHUNK, dtype=np.int64).reshape(NCHUNK, CHUNK)
    bl, fl = flat // F, flat % F
    d0 = jnp.asarray(((bl >> 3) << 8) + ((fl >> 3) << 6)
                     + ((bl & 7) << 3) + (fl & 7), jnp.int32)
    rows = _sc_gather(idx, d0, tbl)
    emb4 = jnp.reshape(rows, (B // 8, NSLAB, 8, 128))
    did3 = domain_id.astype(jnp.int32).reshape(NT, 1, TB)
    W1e = W1[:, :EMB, :]
    W1d = W1[:, EMB:, :]
    W3r = W3[:, :, 0]
    b3r = b3[:, 0]
    return _tc_mlp(emb4, did3, domain_table, W1e, W1d, b1, W2, b2, W3r, b3r,
                   Wd1, bd1, Wd2, bd2, Wo, bo)
